# baseline (speedup 1.0000x reference)
import functools

import jax
import jax.numpy as jnp
from jax import lax
from jax.experimental import pallas as pl
from jax.experimental.pallas import tpu as pltpu
from jax.experimental.pallas import tpu_sc as plsc

_N_IN = 2048
_N_OUT = 1024
_NC = 2
_NS = 16
_NW = _NC * _NS
_L = 16
_GRP = 16
_NQ = 4


@functools.lru_cache(maxsize=None)
def _gather_call(n_rows):
    n_grp = n_rows // _GRP
    mesh = plsc.VectorSubcoreMesh(core_axis_name="c", subcore_axis_name="s")

    @functools.partial(
        pl.kernel,
        mesh=mesh,
        out_type=jax.ShapeDtypeStruct((n_rows, _N_OUT), jnp.float32),
        scratch_types=[
            pltpu.VMEM((_N_OUT,), jnp.int32),
            pltpu.VMEM((_GRP, _N_IN), jnp.float32),
            pltpu.VMEM((_GRP, _N_IN), jnp.float32),
            pltpu.VMEM((_GRP, _N_OUT), jnp.float32),
            pltpu.VMEM((_GRP, _N_OUT), jnp.float32),
            pltpu.SemaphoreType.DMA,
            pltpu.SemaphoreType.DMA,
            pltpu.SemaphoreType.DMA,
            pltpu.SemaphoreType.DMA,
        ],
        compiler_params=pltpu.CompilerParams(needs_layout_passes=False),
    )
    def k(y_hbm, idx_hbm, z_hbm, idx_v, ib0, ib1, ob0, ob1,
          si0, si1, so0, so1):
        wid = lax.axis_index("s") * _NC + lax.axis_index("c")
        g_lo = wid * n_grp // _NW
        g_hi = (wid + 1) * n_grp // _NW
        n_ch = g_hi - g_lo
        pltpu.sync_copy(idx_hbm, idx_v)

        ibufs, obufs = (ib0, ib1), (ob0, ob1)
        isems, osems = (si0, si1), (so0, so1)

        def in_copy(g, b):
            base = pl.multiple_of(g * _GRP, _GRP)
            return pltpu.make_async_copy(
                y_hbm.at[pl.ds(base, _GRP)], ibufs[b], isems[b])

        def out_copy(g, b):
            base = pl.multiple_of(g * _GRP, _GRP)
            return pltpu.make_async_copy(
                obufs[b], z_hbm.at[pl.ds(base, _GRP)], osems[b])

        def compute(ib, ob):
            for q in range(_NQ):
                q0 = q * (_N_OUT // _NQ)
                ivecs = [idx_v[pl.ds(q0 + v * _L, _L)]
                         for v in range(_N_OUT // _NQ // _L)]

                @plsc.parallel_loop(0, _GRP, 1, unroll=8)
                def _(r):
                    rows = jnp.full((_L,), r, jnp.int32)
                    for v, col in enumerate(ivecs):
                        ob[r, pl.ds(q0 + v * _L, _L)] = (
                            plsc.load_gather(ib, [rows, col]))

        in_copy(g_lo, 0).start()

        @pl.when(n_ch > 1)
        def _():
            in_copy(g_lo + 1, 1).start()

        def step(i, b):
            g = g_lo + i

            @pl.when(i < n_ch)
            def _():
                in_copy(g, b).wait()

                @pl.when(i >= 2)
                def _():
                    out_copy(g - 2, b).wait()

                compute(ibufs[b], obufs[b])
                out_copy(g, b).start()

                @pl.when(i + 2 < n_ch)
                def _():
                    in_copy(g + 2, b).start()

        def pair_body(p, _):
            step(2 * p, 0)
            step(2 * p + 1, 1)
            return 0

        lax.fori_loop(0, (n_ch + 1) // 2, pair_body, 0, unroll=False)

        last_par = (n_ch - 1) % 2

        @pl.when((n_ch >= 2) & (last_par == 1))
        def _():
            out_copy(g_hi - 2, 0).wait()

        @pl.when((n_ch >= 2) & (last_par == 0))
        def _():
            out_copy(g_hi - 2, 1).wait()

        @pl.when(last_par == 0)
        def _():
            out_copy(g_hi - 1, 0).wait()

        @pl.when(last_par == 1)
        def _():
            out_copy(g_hi - 1, 1).wait()

    return k


def kernel(x):
    n, c, h, w = x.shape
    ck = jax.random.key(42)
    choices = jax.random.choice(ck, n, shape=(_N_OUT,), replace=False)
    choices = jnp.sort(choices).astype(jnp.int32)
    y = x.transpose(2, 3, 1, 0).reshape(h * w * c, n)
    z = _gather_call(h * w * c)(y, choices)
    return z.reshape(h, w, c, _N_OUT).transpose(3, 2, 0, 1)


# speedup vs baseline: 1.0557x; 1.0557x over previous
import functools

import jax
import jax.numpy as jnp
from jax import lax
from jax.experimental import pallas as pl
from jax.experimental.pallas import tpu as pltpu
from jax.experimental.pallas import tpu_sc as plsc

_N_IN = 2048
_N_OUT = 1024
_NC = 2
_NS = 16
_NW = _NC * _NS
_L = 16
_GRP = 8
_NB = 3
_NQ = 4


@functools.lru_cache(maxsize=None)
def _gather_call(n_rows):
    n_grp = n_rows // _GRP
    mesh = plsc.VectorSubcoreMesh(core_axis_name="c", subcore_axis_name="s")

    @functools.partial(
        pl.kernel,
        mesh=mesh,
        out_type=jax.ShapeDtypeStruct((n_rows, _N_OUT), jnp.float32),
        scratch_types=[
            pltpu.VMEM((_N_OUT,), jnp.int32),
            pltpu.VMEM((_GRP, _N_IN), jnp.float32),
            pltpu.VMEM((_GRP, _N_IN), jnp.float32),
            pltpu.VMEM((_GRP, _N_IN), jnp.float32),
            pltpu.VMEM((_GRP, _N_OUT), jnp.float32),
            pltpu.VMEM((_GRP, _N_OUT), jnp.float32),
            pltpu.VMEM((_GRP, _N_OUT), jnp.float32),
            pltpu.SemaphoreType.DMA,
            pltpu.SemaphoreType.DMA,
            pltpu.SemaphoreType.DMA,
            pltpu.SemaphoreType.DMA,
            pltpu.SemaphoreType.DMA,
            pltpu.SemaphoreType.DMA,
        ],
        compiler_params=pltpu.CompilerParams(needs_layout_passes=False),
    )
    def k(y_hbm, idx_hbm, z_hbm, idx_v, ib0, ib1, ib2, ob0, ob1, ob2,
          si0, si1, si2, so0, so1, so2):
        wid = lax.axis_index("s") * _NC + lax.axis_index("c")
        g_lo = wid * n_grp // _NW
        g_hi = (wid + 1) * n_grp // _NW
        n_ch = g_hi - g_lo
        pltpu.sync_copy(idx_hbm, idx_v)

        ibufs, obufs = (ib0, ib1, ib2), (ob0, ob1, ob2)
        isems, osems = (si0, si1, si2), (so0, so1, so2)

        def in_copy(g, b):
            base = pl.multiple_of(g * _GRP, _GRP)
            return pltpu.make_async_copy(
                y_hbm.at[pl.ds(base, _GRP)], ibufs[b], isems[b])

        def out_copy(g, b):
            base = pl.multiple_of(g * _GRP, _GRP)
            return pltpu.make_async_copy(
                obufs[b], z_hbm.at[pl.ds(base, _GRP)], osems[b])

        def compute(ib, ob):
            for q in range(_NQ):
                q0 = q * (_N_OUT // _NQ)
                ivecs = [idx_v[pl.ds(q0 + v * _L, _L)]
                         for v in range(_N_OUT // _NQ // _L)]

                @plsc.parallel_loop(0, _GRP, 1, unroll=8)
                def _(r):
                    rows = jnp.full((_L,), r, jnp.int32)
                    for v, col in enumerate(ivecs):
                        ob[r, pl.ds(q0 + v * _L, _L)] = (
                            plsc.load_gather(ib, [rows, col]))

        for b in range(_NB - 1):
            in_copy(g_lo + b, b).start()

        def step(i, b):
            g = g_lo + i

            @pl.when(i < n_ch)
            def _():
                in_copy(g, b).wait()

                @pl.when(i + _NB - 1 < n_ch)
                def _():
                    in_copy(g + _NB - 1, (b + _NB - 1) % _NB).start()

                @pl.when(i >= _NB)
                def _():
                    out_copy(g - _NB, b).wait()

                compute(ibufs[b], obufs[b])
                out_copy(g, b).start()

        def round_body(p, _):
            for sub in range(_NB):
                step(_NB * p + sub, sub)
            return 0

        lax.fori_loop(0, (n_ch + _NB - 1) // _NB, round_body, 0,
                      unroll=False)

        for d in range(1, _NB + 1):
            for par in range(_NB):
                @pl.when((n_ch - d) % _NB == par)
                def _(d=d, par=par):
                    out_copy(g_hi - d, par).wait()

    return k


def kernel(x):
    n, c, h, w = x.shape
    ck = jax.random.key(42)
    choices = jax.random.choice(ck, n, shape=(_N_OUT,), replace=False)
    choices = jnp.sort(choices).astype(jnp.int32)
    y = x.transpose(2, 3, 1, 0).reshape(h * w * c, n)
    z = _gather_call(h * w * c)(y, choices)
    return z.reshape(h, w, c, _N_OUT).transpose(3, 2, 0, 1)


# speedup vs baseline: 1.0842x; 1.0270x over previous
import functools

import jax
import jax.numpy as jnp
from jax import lax
from jax.experimental import pallas as pl
from jax.experimental.pallas import tpu as pltpu
from jax.experimental.pallas import tpu_sc as plsc

_N_IN = 2048
_N_OUT = 1024
_NC = 2
_NS = 16
_NW = _NC * _NS
_L = 16
_GRP = 8
_NQ = 4


@functools.lru_cache(maxsize=None)
def _gather_call(n_rows):
    n_grp = n_rows // _GRP
    mesh = plsc.VectorSubcoreMesh(core_axis_name="c", subcore_axis_name="s")

    @functools.partial(
        pl.kernel,
        mesh=mesh,
        out_type=jax.ShapeDtypeStruct((n_rows, _N_OUT), jnp.float32),
        scratch_types=[
            pltpu.VMEM((_N_OUT,), jnp.int32),
            pltpu.VMEM((_GRP, _N_IN), jnp.float32),
            pltpu.VMEM((_GRP, _N_IN), jnp.float32),
            pltpu.VMEM((_GRP, _N_OUT), jnp.float32),
            pltpu.VMEM((_GRP, _N_OUT), jnp.float32),
            pltpu.SemaphoreType.DMA,
            pltpu.SemaphoreType.DMA,
            pltpu.SemaphoreType.DMA,
            pltpu.SemaphoreType.DMA,
        ],
        compiler_params=pltpu.CompilerParams(
            needs_layout_passes=False, skip_device_barrier=True),
    )
    def k(y_hbm, idx_hbm, z_hbm, idx_v, ib0, ib1, ob0, ob1,
          si0, si1, so0, so1):
        wid = lax.axis_index("s") * _NC + lax.axis_index("c")
        g_lo = wid * n_grp // _NW
        g_hi = (wid + 1) * n_grp // _NW
        n_ch = g_hi - g_lo
        pltpu.sync_copy(idx_hbm, idx_v)

        ibufs, obufs = (ib0, ib1), (ob0, ob1)
        isems, osems = (si0, si1), (so0, so1)

        def in_copy(g, b):
            base = pl.multiple_of(g * _GRP, _GRP)
            return pltpu.make_async_copy(
                y_hbm.at[pl.ds(base, _GRP)], ibufs[b], isems[b])

        def out_copy(g, b):
            base = pl.multiple_of(g * _GRP, _GRP)
            return pltpu.make_async_copy(
                obufs[b], z_hbm.at[pl.ds(base, _GRP)], osems[b])

        def compute(ib, ob):
            for q in range(_NQ):
                q0 = q * (_N_OUT // _NQ)
                ivecs = [idx_v[pl.ds(q0 + v * _L, _L)]
                         for v in range(_N_OUT // _NQ // _L)]

                @plsc.parallel_loop(0, _GRP, 1, unroll=8)
                def _(r):
                    rows = jnp.full((_L,), r, jnp.int32)
                    for v, col in enumerate(ivecs):
                        ob[r, pl.ds(q0 + v * _L, _L)] = (
                            plsc.load_gather(ib, [rows, col]))

        in_copy(g_lo, 0).start()
        in_copy(g_lo + 1, 1).start()

        def step(i, b):
            g = g_lo + i

            @pl.when(i < n_ch)
            def _():
                in_copy(g, b).wait()

                @pl.when(i >= 2)
                def _():
                    out_copy(g - 2, b).wait()

                compute(ibufs[b], obufs[b])
                out_copy(g, b).start()

                @pl.when(i + 2 < n_ch)
                def _():
                    in_copy(g + 2, b).start()

        def pair_body(p, _):
            step(2 * p, 0)
            step(2 * p + 1, 1)
            return 0

        lax.fori_loop(0, (n_ch + 1) // 2, pair_body, 0, unroll=False)

        last_par = (n_ch - 1) % 2

        @pl.when(last_par == 1)
        def _():
            out_copy(g_hi - 2, 0).wait()

        @pl.when(last_par == 0)
        def _():
            out_copy(g_hi - 2, 1).wait()

        @pl.when(last_par == 0)
        def _():
            out_copy(g_hi - 1, 0).wait()

        @pl.when(last_par == 1)
        def _():
            out_copy(g_hi - 1, 1).wait()

    return k


def kernel(x):
    n, c, h, w = x.shape
    ck = jax.random.key(42)
    choices = jax.random.choice(ck, n, shape=(_N_OUT,), replace=False)
    choices = jnp.sort(choices).astype(jnp.int32)
    y = x.transpose(2, 3, 1, 0).reshape(h * w * c, n)
    z = _gather_call(h * w * c)(y, choices)
    return z.reshape(h, w, c, _N_OUT).transpose(3, 2, 0, 1)
